# baseline (speedup 1.0000x reference)
import functools

import jax
import jax.numpy as jnp
from jax import lax
from jax.experimental import pallas as pl
from jax.experimental.pallas import tpu as pltpu
from jax.experimental.pallas import tpu_sc as plsc

B, C, U, F = 4, 24, 16, 4
E, H, DS, DE = 128, 128, 64, 64
VW, VS, VE = 100000, 10, 8
SH = E // 2
BC = B * C
NARC = B * C * C
NIDX = B * C * U * F


def _sc_gather(table, idx):
    info = plsc.get_sparse_core_info()
    nc, ns = info.num_cores, info.num_subcores
    nw = nc * ns
    bpw = NIDX // nw
    mesh = plsc.VectorSubcoreMesh(core_axis_name="c", subcore_axis_name="s")

    @functools.partial(
        pl.kernel,
        mesh=mesh,
        out_type=jax.ShapeDtypeStruct((NIDX, E), jnp.float32),
        scratch_types=[
            pltpu.VMEM((bpw,), jnp.int32),
            pltpu.VMEM((bpw, E), jnp.float32),
            pltpu.SemaphoreType.DMA,
        ],
    )
    def k(table_hbm, idx_hbm, out_hbm, idx_v, rows_v, sem):
        wid = lax.axis_index("s") * nc + lax.axis_index("c")
        base = wid * bpw
        pltpu.sync_copy(idx_hbm.at[pl.ds(base, bpw)], idx_v)
        pltpu.async_copy(table_hbm.at[idx_v], rows_v, sem).wait()
        pltpu.sync_copy(rows_v, out_hbm.at[pl.ds(base, bpw)])

    return k(table, idx)


def _sig(x):
    return 0.5 * jnp.tanh(0.5 * x) + 0.5


def _dot(a, b):
    return jax.lax.dot_general(a, b, (((1,), (0,)), ((), ())),
                               preferred_element_type=jnp.float32)


def _fused_body(g_ref, spk_ids_ref, em_ids_ref, gcol_ref,
                utAf_ref, utAb_ref, utR2_ref, utb2_ref,
                spk_tab_ref, em_tab_ref,
                wc_ref, wcb_ref, we_ref, web_ref,
                emc_ref, emcb_ref, eme_ref, emeb_ref,
                wut_ref, wem_ref,
                spAf_ref, spAb_ref, spE2_ref, spb2_ref,
                spR2_ref, w2_ref, ob_ref,
                sut_ref, sem_ref, sp_ref):
    g = g_ref[...]
    we = (g[0] + g[1] + g[2] + g[3]) * 0.25
    we3 = we.reshape(U, BC, E)
    xs = [we3[t] for t in range(U)]

    utAf = utAf_ref[...]
    utAb = utAb_ref[...]
    pf = [_dot(xs[t], utAf) for t in range(U)]
    pb = [_dot(xs[t], utAb) for t in range(U)]

    def pack_ut(a, b):
        return jnp.concatenate([
            a[:, 0:128], b[:, 0:128],
            a[:, 128:256], b[:, 128:256],
            a[:, 384:512], b[:, 384:512],
            a[:, 256:384], b[:, 256:384],
        ], axis=1)

    utR2 = utR2_ref[...]
    utb2 = utb2_ref[...]
    h2 = jnp.zeros((BC, 2 * H), jnp.float32)
    c2 = jnp.zeros((BC, 2 * H), jnp.float32)
    for s in range(U):
        gates = pack_ut(pf[s], pb[U - 1 - s]) + utb2 + _dot(h2, utR2)
        sg = _sig(gates[:, 0:768])
        gg = jnp.tanh(gates[:, 768:1024])
        c2 = sg[:, 256:512] * c2 + sg[:, 0:256] * gg
        h2 = sg[:, 512:768] * jnp.tanh(c2)

    spk_oh = (spk_ids_ref[...] ==
              jax.lax.broadcasted_iota(jnp.int32, (BC, VS), 1)
              ).astype(jnp.float32)
    spk = _dot(spk_oh, spk_tab_ref[...])
    em_oh = (em_ids_ref[...] ==
             jax.lax.broadcasted_iota(jnp.int32, (BC, VE), 1)
             ).astype(jnp.float32)
    em_e = _dot(em_oh, em_tab_ref[...])

    ut = jnp.concatenate([h2, spk], axis=-1)

    def ffn(wref, bref):
        y = _dot(ut, wref[...]) + bref[...]
        return jnp.where(y >= 0, y, 0.1 * y)

    ut_cause = ffn(wc_ref, wcb_ref)
    ut_effect = ffn(we_ref, web_ref)
    em_cause = ffn(emc_ref, emcb_ref)
    em_effect = ffn(eme_ref, emeb_ref)

    ones = jnp.ones((C, 1), jnp.float32)
    wut = wut_ref[...]
    wem = wem_ref[...]
    t1_rows = []
    for bb in range(B):
        r0 = bb * C
        xe = jnp.concatenate([ut_effect[r0:r0 + C], ones], axis=-1)
        yc = ut_cause[r0:r0 + C]
        t1 = _dot(xe, wut)
        t1_rows.append(t1)
        sut_ref[bb] = jax.lax.dot_general(t1, yc, (((1,), (1,)), ((), ())),
                                          preferred_element_type=jnp.float32)
        xem = jnp.concatenate([em_effect[r0:r0 + C], ones], axis=-1)
        yem = jnp.concatenate([em_cause[r0:r0 + C], ones], axis=-1)
        for o in range(VE):
            t2 = _dot(xem, wem[o])
            sem_ref[bb, o] = jax.lax.dot_general(
                t2, yem, (((1,), (1,)), ((), ())),
                preferred_element_type=jnp.float32)

    def expand_cau(x):
        w = x.shape[1]
        x4 = x.reshape(B, 1, C, w)
        return jnp.broadcast_to(x4, (B, C, C, w)).reshape(NARC, w)

    def expand_eff(x):
        w = x.shape[1]
        x4 = x.reshape(B, C, 1, w)
        return jnp.broadcast_to(x4, (B, C, C, w)).reshape(NARC, w)

    spAf = spAf_ref[...]
    spAb = spAb_ref[...]
    xw2 = [_dot(xs[s], spAf) + _dot(xs[U - 1 - s], spAb) for s in range(U)]
    eb2 = expand_eff(_dot(em_e, spE2_ref[...]) + spb2_ref[...])

    spR2 = spR2_ref[...]
    w2 = w2_ref[...]
    h2s = jnp.zeros((NARC, 2 * SH), jnp.float32)
    c2s = jnp.zeros((NARC, 2 * SH), jnp.float32)
    fcol = [None] * U
    bcol = [None] * U
    for s in range(U):
        gates = expand_cau(xw2[s]) + eb2 + _dot(h2s, spR2)
        sg = _sig(gates[:, 0:384])
        gg = jnp.tanh(gates[:, 384:512])
        c2s = sg[:, 128:256] * c2s + sg[:, 0:128] * gg
        h2s = sg[:, 256:384] * jnp.tanh(c2s)
        p2 = _dot(h2s, w2)
        fcol[s] = p2[:, 0:1]
        bcol[U - 1 - s] = p2[:, 1:2]

    logit = jnp.concatenate([fcol[t] + bcol[t] for t in range(U)],
                            axis=1) + ob_ref[...]
    preds = _sig(logit)

    t196 = jnp.concatenate(t1_rows, axis=0)
    s_col = jnp.sum(expand_eff(t196) * expand_cau(ut_cause),
                    axis=1, keepdims=True)
    mask = (gcol_ref[...] != 0) | (s_col > 0.0)
    sp_ref[...] = preds * mask.astype(jnp.float32)


def _pack_cols(mT, width, pos, total):
    k = mT.shape[0]
    out = jnp.zeros((k, total), jnp.float32)
    out = out.at[:, pos + 0 * 2 * width:pos + 0 * 2 * width + width].set(
        mT[:, 0:width])
    out = out.at[:, pos + 1 * 2 * width:pos + 1 * 2 * width + width].set(
        mT[:, width:2 * width])
    out = out.at[:, pos + 2 * 2 * width:pos + 2 * 2 * width + width].set(
        mT[:, 3 * width:4 * width])
    out = out.at[:, pos + 3 * 2 * width:pos + 3 * 2 * width + width].set(
        mT[:, 2 * width:3 * width])
    return out


def _fused_call(g, spk_ids, em_ids, gcol, p):
    utRf = _pack_cols(p['ut_Whh_f'].T, H, 0, 8 * H)
    utRb = _pack_cols(p['ut_Whh_b'].T, H, H, 8 * H)
    utR2 = jnp.concatenate([utRf, utRb], axis=0)
    utb2 = (_pack_cols(p['ut_b_f'].reshape(1, -1), H, 0, 8 * H)
            + _pack_cols(p['ut_b_b'].reshape(1, -1), H, H, 8 * H))

    spAf = _pack_cols(p['sp_Wih_f'][:, :E].T, SH, 0, 8 * SH)
    spAb = _pack_cols(p['sp_Wih_b'][:, :E].T, SH, SH, 8 * SH)
    spE2 = (_pack_cols(p['sp_Wih_f'][:, E:].T, SH, 0, 8 * SH)
            + _pack_cols(p['sp_Wih_b'][:, E:].T, SH, SH, 8 * SH))
    spb2 = (_pack_cols(p['sp_b_f'].reshape(1, -1), SH, 0, 8 * SH)
            + _pack_cols(p['sp_b_b'].reshape(1, -1), SH, SH, 8 * SH))
    spRf = _pack_cols(p['sp_Whh_f'].T, SH, 0, 8 * SH)
    spRb = _pack_cols(p['sp_Whh_b'].T, SH, SH, 8 * SH)
    spR2 = jnp.concatenate([spRf, spRb], axis=0)
    w2 = jnp.zeros((2 * SH, 2), jnp.float32)
    w2 = w2.at[0:SH, 0].set(p['sp_out_W'][0, 0:SH])
    w2 = w2.at[SH:2 * SH, 1].set(p['sp_out_W'][0, SH:2 * SH])

    out_shapes = [
        jax.ShapeDtypeStruct((B, C, C), jnp.float32),
        jax.ShapeDtypeStruct((B, VE, C, C), jnp.float32),
        jax.ShapeDtypeStruct((NARC, U), jnp.float32),
    ]
    args = [
        g.reshape(F, U * BC, E),
        spk_ids, em_ids, gcol,
        p['ut_Wih_f'].T, p['ut_Wih_b'].T, utR2, utb2,
        p['spk_table'], p['em_table'],
        p['ut_cause_W'].T, p['ut_cause_b'].reshape(1, -1),
        p['ut_effect_W'].T, p['ut_effect_b'].reshape(1, -1),
        p['em_cause_W'].T, p['em_cause_b'].reshape(1, -1),
        p['em_effect_W'].T, p['em_effect_b'].reshape(1, -1),
        p['W_ut'][0], p['W_em'],
        spAf, spAb, spE2, spb2,
        spR2, w2, p['sp_out_b'].reshape(1, 1),
    ]
    return pl.pallas_call(_fused_body, out_shape=out_shapes)(*args)


def kernel(words, speakers, emotions, graphs, spans, params):
    del spans
    idx = words.astype(jnp.int32).transpose(3, 2, 0, 1).reshape(-1)
    g = _sc_gather(params['word_table'], idx)

    spk_ids = speakers.astype(jnp.int32).reshape(BC, 1)
    em_ids = emotions.astype(jnp.int32).reshape(BC, 1)
    gcol = graphs.astype(jnp.int32).reshape(NARC, 1)
    s_ut, s_em_k, sp = _fused_call(g, spk_ids, em_ids, gcol, params)

    s_em = jnp.transpose(s_em_k, (0, 2, 3, 1))
    s_span = sp.reshape(B, C, C, U)
    return (s_ut, s_em, s_span)


# speedup vs baseline: 1.5902x; 1.5902x over previous
import functools

import jax
import jax.numpy as jnp
from jax import lax
from jax.experimental import pallas as pl
from jax.experimental.pallas import tpu as pltpu
from jax.experimental.pallas import tpu_sc as plsc

B, C, U, F = 4, 24, 16, 4
E, H, DS, DE = 128, 128, 64, 64
VW, VS, VE = 100000, 10, 8
SH = E // 2
BC = B * C
NARC = B * C * C
NIDX = B * C * U * F


def _sc_gather(table, idx):
    info = plsc.get_sparse_core_info()
    nc, ns = info.num_cores, info.num_subcores
    nw = nc * ns
    bpw = NIDX // nw
    mesh = plsc.VectorSubcoreMesh(core_axis_name="c", subcore_axis_name="s")

    @functools.partial(
        pl.kernel,
        mesh=mesh,
        out_type=jax.ShapeDtypeStruct((NIDX, E), jnp.float32),
        scratch_types=[
            pltpu.VMEM((bpw,), jnp.int32),
            pltpu.VMEM((bpw, E), jnp.float32),
            pltpu.SemaphoreType.DMA,
        ],
    )
    def k(table_hbm, idx_hbm, out_hbm, idx_v, rows_v, sem):
        wid = lax.axis_index("s") * nc + lax.axis_index("c")
        base = wid * bpw
        pltpu.sync_copy(idx_hbm.at[pl.ds(base, bpw)], idx_v)
        pltpu.async_copy(table_hbm.at[idx_v], rows_v, sem).wait()
        pltpu.sync_copy(rows_v, out_hbm.at[pl.ds(base, bpw)])

    return k(table, idx)


def _sig(x):
    return 0.5 * jnp.tanh(0.5 * x) + 0.5


def _dot(a, b):
    return jax.lax.dot_general(a, b, (((1,), (0,)), ((), ())),
                               preferred_element_type=jnp.float32)


def _dot_t(a, b):
    return jax.lax.dot_general(a, b, (((1,), (1,)), ((), ())),
                               preferred_element_type=jnp.float32)


_GATE_ORDER = (0, 1, 3, 2)


def _pack_rec(mf, mb, w, z):
    parts = []
    for gidx in _GATE_ORDER:
        lo = gidx * w
        parts.append(jnp.concatenate([mf[lo:lo + w], z], axis=1))
        parts.append(jnp.concatenate([z, mb[lo:lo + w]], axis=1))
    return jnp.concatenate(parts, axis=0)


def _pack_rows(mf, mb, w):
    parts = []
    for gidx in _GATE_ORDER:
        lo = gidx * w
        parts.append(mf[lo:lo + w])
        parts.append(mb[lo:lo + w])
    return jnp.concatenate(parts, axis=0)


def _pack_half(m, w, z, fwd_live):
    parts = []
    for gidx in _GATE_ORDER:
        lo = gidx * w
        if fwd_live:
            parts.append(m[lo:lo + w])
            parts.append(z)
        else:
            parts.append(z)
            parts.append(m[lo:lo + w])
    return jnp.concatenate(parts, axis=0)


def _fused_body(g_ref, spk_ids_ref, em_ids_ref, gcol_ref,
                utWihf_ref, utWihb_ref, utWhhf_ref, utWhhb_ref,
                utbf_ref, utbb_ref,
                spk_tab_ref, em_tab_ref,
                wc_ref, wcb_ref, we_ref, web_ref,
                emc_ref, emcb_ref, eme_ref, emeb_ref,
                wut_ref, wem_ref,
                spWihf_ref, spWihb_ref, spWhhf_ref, spWhhb_ref,
                spbf_ref, spbb_ref, spow_ref, spob_ref,
                sut_ref, sem_ref, sp_ref):
    g = g_ref[...]
    we_all = (g[0] + g[1] + g[2] + g[3]) * 0.25
    we3 = we_all.reshape(U, BC, E)
    xs = [we3[t] for t in range(U)]

    utWihf = utWihf_ref[...]
    utWihb = utWihb_ref[...]
    pf = [_dot_t(xs[t], utWihf) for t in range(U)]
    pb = [_dot_t(xs[t], utWihb) for t in range(U)]

    def pack_cols_ut(a, b):
        return jnp.concatenate([
            a[:, 0:H], b[:, 0:H],
            a[:, H:2 * H], b[:, H:2 * H],
            a[:, 3 * H:4 * H], b[:, 3 * H:4 * H],
            a[:, 2 * H:3 * H], b[:, 2 * H:3 * H],
        ], axis=1)

    zH = jnp.zeros((H, H), jnp.float32)
    utR2 = _pack_rec(utWhhf_ref[...], utWhhb_ref[...], H, zH)
    utb2 = pack_cols_ut(utbf_ref[...], utbb_ref[...])

    h2 = jnp.zeros((BC, 2 * H), jnp.float32)
    c2 = jnp.zeros((BC, 2 * H), jnp.float32)
    for s in range(U):
        gates = pack_cols_ut(pf[s], pb[U - 1 - s]) + utb2 + _dot_t(h2, utR2)
        sg = _sig(gates[:, 0:768])
        gg = jnp.tanh(gates[:, 768:1024])
        c2 = sg[:, 256:512] * c2 + sg[:, 0:256] * gg
        h2 = sg[:, 512:768] * jnp.tanh(c2)

    spk_oh = (spk_ids_ref[...] ==
              jax.lax.broadcasted_iota(jnp.int32, (BC, VS), 1)
              ).astype(jnp.float32)
    spk = _dot(spk_oh, spk_tab_ref[...])
    em_oh = (em_ids_ref[...] ==
             jax.lax.broadcasted_iota(jnp.int32, (BC, VE), 1)
             ).astype(jnp.float32)
    em_e = _dot(em_oh, em_tab_ref[...])

    ut = jnp.concatenate([h2, spk], axis=-1)

    def ffn(wref, bref):
        y = _dot_t(ut, wref[...]) + bref[...]
        return jnp.where(y >= 0, y, 0.1 * y)

    ut_cause = ffn(wc_ref, wcb_ref)
    ut_effect = ffn(we_ref, web_ref)
    em_cause = ffn(emc_ref, emcb_ref)
    em_effect = ffn(eme_ref, emeb_ref)

    ones = jnp.ones((C, 1), jnp.float32)
    wut = wut_ref[0]
    wem = wem_ref[...]
    t1_rows = []
    for bb in range(B):
        r0 = bb * C
        xe = jnp.concatenate([ut_effect[r0:r0 + C], ones], axis=-1)
        yc = ut_cause[r0:r0 + C]
        t1 = _dot(xe, wut)
        t1_rows.append(t1)
        sut_ref[bb] = _dot_t(t1, yc)
        xem = jnp.concatenate([em_effect[r0:r0 + C], ones], axis=-1)
        yem = jnp.concatenate([em_cause[r0:r0 + C], ones], axis=-1)
        for o in range(VE):
            t2 = _dot(xem, wem[o])
            sem_ref[bb, o] = _dot_t(t2, yem)

    def expand_cau(x):
        w = x.shape[1]
        x4 = x.reshape(B, 1, C, w)
        return jnp.broadcast_to(x4, (B, C, C, w)).reshape(NARC, w)

    def expand_eff(x):
        w = x.shape[1]
        x4 = x.reshape(B, C, 1, w)
        return jnp.broadcast_to(x4, (B, C, C, w)).reshape(NARC, w)

    spWihf = spWihf_ref[...]
    spWihb = spWihb_ref[...]
    zS = jnp.zeros((SH, E), jnp.float32)
    spAf = _pack_half(spWihf[:, 0:E], SH, zS, True)
    spAb = _pack_half(spWihb[:, 0:E], SH, zS, False)
    spE2 = _pack_rows(spWihf[:, E:], spWihb[:, E:], SH)

    def pack_cols_sp(a, b):
        return jnp.concatenate([
            a[:, 0:SH], b[:, 0:SH],
            a[:, SH:2 * SH], b[:, SH:2 * SH],
            a[:, 3 * SH:4 * SH], b[:, 3 * SH:4 * SH],
            a[:, 2 * SH:3 * SH], b[:, 2 * SH:3 * SH],
        ], axis=1)

    spb2 = pack_cols_sp(spbf_ref[...], spbb_ref[...])
    zSh = jnp.zeros((SH, SH), jnp.float32)
    spR2 = _pack_rec(spWhhf_ref[...], spWhhb_ref[...], SH, zSh)

    xw2 = [_dot_t(xs[s], spAf) + _dot_t(xs[U - 1 - s], spAb)
           for s in range(U)]
    eb2 = expand_eff(_dot_t(em_e, spE2) + spb2)

    spow = spow_ref[...]
    z1 = jnp.zeros((1, SH), jnp.float32)
    w2 = jnp.concatenate([
        jnp.concatenate([spow[:, 0:SH], z1], axis=1),
        jnp.concatenate([z1, spow[:, SH:2 * SH]], axis=1),
    ], axis=0)

    h2s = jnp.zeros((NARC, 2 * SH), jnp.float32)
    c2s = jnp.zeros((NARC, 2 * SH), jnp.float32)
    fcol = [None] * U
    bcol = [None] * U
    for s in range(U):
        gates = expand_cau(xw2[s]) + eb2 + _dot_t(h2s, spR2)
        sg = _sig(gates[:, 0:384])
        gg = jnp.tanh(gates[:, 384:512])
        c2s = sg[:, 128:256] * c2s + sg[:, 0:128] * gg
        h2s = sg[:, 256:384] * jnp.tanh(c2s)
        p2 = _dot_t(h2s, w2)
        fcol[s] = p2[:, 0:1]
        bcol[U - 1 - s] = p2[:, 1:2]

    logit = jnp.concatenate([fcol[t] + bcol[t] for t in range(U)],
                            axis=1) + spob_ref[...]
    preds = _sig(logit)

    t196 = jnp.concatenate(t1_rows, axis=0)
    s_col = jnp.sum(expand_eff(t196) * expand_cau(ut_cause),
                    axis=1, keepdims=True)
    mask = (gcol_ref[...] != 0) | (s_col > 0.0)
    sp_ref[...] = preds * mask.astype(jnp.float32)


def _fused_call(g, spk_ids, em_ids, gcol, p):
    out_shapes = [
        jax.ShapeDtypeStruct((B, C, C), jnp.float32),
        jax.ShapeDtypeStruct((B, VE, C, C), jnp.float32),
        jax.ShapeDtypeStruct((NARC, U), jnp.float32),
    ]
    args = [
        g.reshape(F, U * BC, E),
        spk_ids, em_ids, gcol,
        p['ut_Wih_f'], p['ut_Wih_b'], p['ut_Whh_f'], p['ut_Whh_b'],
        p['ut_b_f'].reshape(1, -1), p['ut_b_b'].reshape(1, -1),
        p['spk_table'], p['em_table'],
        p['ut_cause_W'], p['ut_cause_b'].reshape(1, -1),
        p['ut_effect_W'], p['ut_effect_b'].reshape(1, -1),
        p['em_cause_W'], p['em_cause_b'].reshape(1, -1),
        p['em_effect_W'], p['em_effect_b'].reshape(1, -1),
        p['W_ut'], p['W_em'],
        p['sp_Wih_f'], p['sp_Wih_b'], p['sp_Whh_f'], p['sp_Whh_b'],
        p['sp_b_f'].reshape(1, -1), p['sp_b_b'].reshape(1, -1),
        p['sp_out_W'], p['sp_out_b'].reshape(1, 1),
    ]
    return pl.pallas_call(_fused_body, out_shape=out_shapes)(*args)


def kernel(words, speakers, emotions, graphs, spans, params):
    del spans
    idx = words.astype(jnp.int32).transpose(3, 2, 0, 1).reshape(-1)
    g = _sc_gather(params['word_table'], idx)

    spk_ids = speakers.astype(jnp.int32).reshape(BC, 1)
    em_ids = emotions.astype(jnp.int32).reshape(BC, 1)
    gcol = graphs.astype(jnp.int32).reshape(NARC, 1)
    s_ut, s_em_k, sp = _fused_call(g, spk_ids, em_ids, gcol, params)

    s_em = jnp.transpose(s_em_k, (0, 2, 3, 1))
    s_span = sp.reshape(B, C, C, U)
    return (s_ut, s_em, s_span)
